# baseline (speedup 1.0000x reference)
import functools

import jax
import jax.numpy as jnp
from jax import lax
from jax.experimental import pallas as pl
from jax.experimental.pallas import tpu as pltpu
from jax.experimental.pallas import tpu_sc as plsc

BATCH = 4096
HIST = 200
EMBED = 128
NUM_EDGE_TYPE = 64
N_ROWS = BATCH * HIST
NUM_WORKERS = 32
ROWS_PER_W = N_ROWS // NUM_WORKERS
CHUNK = 128
N_CHUNKS = ROWS_PER_W // CHUNK
NBUF = 4
GROUPS = CHUNK // 16

_mesh = plsc.VectorSubcoreMesh(core_axis_name="c", subcore_axis_name="s")


@functools.partial(
    pl.kernel,
    mesh=_mesh,
    out_type=jax.ShapeDtypeStruct((N_ROWS, EMBED), jnp.float32),
    compiler_params=pltpu.CompilerParams(needs_layout_passes=False),
    scratch_types=(
        [pltpu.VMEM((NUM_EDGE_TYPE, EMBED), jnp.float32),
         pltpu.VMEM((ROWS_PER_W,), jnp.int32),
         pltpu.VMEM((NBUF * CHUNK, EMBED), jnp.float32)]
        + [pltpu.SemaphoreType.DMA] * NBUF
    ),
)
def _expand(idx_hbm, table_hbm, out_hbm, table_v, idx_v, stage_v, *osems):
    wid = lax.axis_index("s") * 2 + lax.axis_index("c")
    row_base = wid * ROWS_PER_W

    pltpu.sync_copy(table_hbm, table_v)
    pltpu.sync_copy(idx_hbm.at[pl.ds(row_base * 1, ROWS_PER_W)], idx_v)

    iota16 = lax.iota(jnp.int32, 16)

    def odesc(c, b):
        return pltpu.make_async_copy(
            stage_v.at[pl.ds(b * CHUNK, CHUNK)],
            out_hbm.at[pl.ds(row_base + c * CHUNK, CHUNK)],
            osems[b])

    def expand(c, b):
        def qbody(q, carry):
            src_rows = idx_v[pl.ds(c * CHUNK + q * 16, 16)]
            dst_rows = iota16 + (b * CHUNK + q * 16)
            for col in range(EMBED):
                colv = jnp.full((16,), col, jnp.int32)
                v = plsc.load_gather(table_v, [src_rows, colv])
                plsc.store_scatter(stage_v, [dst_rows, colv], v)
            return carry
        lax.fori_loop(0, GROUPS, qbody, 0)

    for b in range(NBUF):
        expand(b, b)
        odesc(b, b).start()

    def body(g, carry):
        cb = NBUF * g + NBUF
        for b in range(NBUF):
            c = cb + b
            odesc(c - NBUF, b).wait()
            expand(c, b)
            odesc(c, b).start()
        return carry

    lax.fori_loop(0, (N_CHUNKS - NBUF) // NBUF, body, 0)
    for b in range(NBUF):
        odesc(N_CHUNKS - NBUF + b, b).wait()


def kernel(data, edge_type_embedding):
    idx = data.reshape(N_ROWS)
    out = _expand(idx, edge_type_embedding)
    return out.reshape(BATCH, HIST, EMBED)


# speedup vs baseline: 1.5990x; 1.5990x over previous
import functools

import jax
import jax.numpy as jnp
from jax import lax
from jax.experimental import pallas as pl
from jax.experimental.pallas import tpu as pltpu
from jax.experimental.pallas import tpu_sc as plsc

BATCH = 4096
HIST = 200
EMBED = 128
NUM_EDGE_TYPE = 64
N_ROWS = BATCH * HIST
NUM_WORKERS = 32
ROWS_PER_W = N_ROWS // NUM_WORKERS
CHUNK = 128
N_CHUNKS = ROWS_PER_W // CHUNK
NBUF = 4
GROUPS = CHUNK // 16

_mesh = plsc.VectorSubcoreMesh(core_axis_name="c", subcore_axis_name="s")


@functools.partial(
    pl.kernel,
    mesh=_mesh,
    out_type=jax.ShapeDtypeStruct((N_ROWS, EMBED), jnp.float32),
    compiler_params=pltpu.CompilerParams(needs_layout_passes=False),
    scratch_types=(
        [pltpu.VMEM((NUM_EDGE_TYPE, EMBED), jnp.float32),
         pltpu.VMEM((ROWS_PER_W,), jnp.int32),
         pltpu.VMEM((NBUF * CHUNK, EMBED), jnp.float32)]
        + [pltpu.SemaphoreType.DMA] * NBUF
    ),
)
def _expand(idx_hbm, table_hbm, out_hbm, table_v, idx_v, stage_v, *osems):
    wid = lax.axis_index("s") * 2 + lax.axis_index("c")
    row_base = wid * ROWS_PER_W

    pltpu.sync_copy(table_hbm, table_v)
    pltpu.sync_copy(idx_hbm.at[pl.ds(row_base * 1, ROWS_PER_W)], idx_v)

    iota16 = lax.iota(jnp.int32, 16)

    def odesc(c, b):
        return pltpu.make_async_copy(
            stage_v.at[pl.ds(b * CHUNK, CHUNK)],
            out_hbm.at[pl.ds(row_base + c * CHUNK, CHUNK)],
            osems[b])

    def expand(c, b):
        def qbody(q, carry):
            src_rows = idx_v[pl.ds(c * CHUNK + q * 16, 16)]
            dst_rows = iota16 + (b * CHUNK + q * 16)

            @plsc.parallel_loop(0, EMBED, unroll=8)
            def colbody(col):
                colv = jnp.broadcast_to(col, (16,))
                v = plsc.load_gather(table_v, [src_rows, colv])
                plsc.store_scatter(stage_v, [dst_rows, colv], v)

            return carry
        lax.fori_loop(0, GROUPS, qbody, 0)

    for b in range(NBUF):
        expand(b, b)
        odesc(b, b).start()

    def body(g, carry):
        cb = NBUF * g + NBUF
        for b in range(NBUF):
            c = cb + b
            odesc(c - NBUF, b).wait()
            expand(c, b)
            odesc(c, b).start()
        return carry

    lax.fori_loop(0, (N_CHUNKS - NBUF) // NBUF, body, 0)
    for b in range(NBUF):
        odesc(N_CHUNKS - NBUF + b, b).wait()


def kernel(data, edge_type_embedding):
    idx = data.reshape(N_ROWS)
    out = _expand(idx, edge_type_embedding)
    return out.reshape(BATCH, HIST, EMBED)


# speedup vs baseline: 8.7349x; 5.4626x over previous
import functools

import jax
import jax.numpy as jnp
from jax import lax
from jax.experimental import pallas as pl
from jax.experimental.pallas import tpu as pltpu
from jax.experimental.pallas import tpu_sc as plsc

BATCH = 4096
HIST = 200
EMBED = 128
NUM_EDGE_TYPE = 64
N_ROWS = BATCH * HIST
NUM_WORKERS = 32
ROWS_PER_W = N_ROWS // NUM_WORKERS
CHUNK = 128
N_CHUNKS = ROWS_PER_W // CHUNK
NBUF = 4
COLB = EMBED // 16

_mesh = plsc.VectorSubcoreMesh(core_axis_name="c", subcore_axis_name="s")


@functools.partial(
    pl.kernel,
    mesh=_mesh,
    out_type=jax.ShapeDtypeStruct((N_ROWS * EMBED,), jnp.float32),
    compiler_params=pltpu.CompilerParams(needs_layout_passes=False),
    scratch_types=(
        [pltpu.VMEM((NUM_EDGE_TYPE * EMBED,), jnp.float32),
         pltpu.VMEM((NBUF * CHUNK,), jnp.int32),
         pltpu.VMEM((NBUF * CHUNK * EMBED,), jnp.float32)]
        + [pltpu.SemaphoreType.DMA] * (2 * NBUF)
    ),
)
def _expand(idx_hbm, table_hbm, out_hbm, table_v, idx_v, stage_v, *sems):
    isems, osems = sems[:NBUF], sems[NBUF:]
    wid = lax.axis_index("s") * 2 + lax.axis_index("c")
    crow_base = wid * N_CHUNKS
    out_base = wid * ROWS_PER_W * EMBED

    pltpu.sync_copy(table_hbm, table_v)

    def idesc(c, b):
        return pltpu.make_async_copy(
            idx_hbm.at[pl.ds((crow_base + c) * CHUNK, CHUNK)],
            idx_v.at[pl.ds(b * CHUNK, CHUNK)],
            isems[b])

    def odesc(c, b):
        return pltpu.make_async_copy(
            stage_v.at[pl.ds(b * CHUNK * EMBED, CHUNK * EMBED)],
            out_hbm.at[pl.ds(out_base + c * CHUNK * EMBED, CHUNK * EMBED)],
            osems[b])

    def expand(b):
        @plsc.parallel_loop(0, CHUNK // 16, unroll=1)
        def qbody(q):
            ivec = idx_v[pl.ds(b * CHUNK + q * 16, 16)]
            dst0 = (b * CHUNK + q * 16) * EMBED
            for l in range(16):
                src = ivec[l] * EMBED
                dst = dst0 + l * EMBED
                for j in range(COLB):
                    stage_v[pl.ds(dst + 16 * j, 16)] = (
                        table_v[pl.ds(src + 16 * j, 16)])

    for b in range(NBUF):
        idesc(b, b).start()

    n_groups = N_CHUNKS // NBUF

    def body(g, carry):
        cb = NBUF * g
        for b in range(NBUF):
            c = cb + b
            pl.when(g > 0)(lambda: odesc(c - NBUF, b).wait())
            idesc(c, b).wait()
            expand(b)
            pl.when(g < n_groups - 1)(lambda: idesc(c + NBUF, b).start())
            odesc(c, b).start()
        return carry

    lax.fori_loop(0, n_groups, body, 0)
    for b in range(NBUF):
        odesc(N_CHUNKS - NBUF + b, b).wait()


def kernel(data, edge_type_embedding):
    idx = data.reshape(N_ROWS)
    table = edge_type_embedding.reshape(NUM_EDGE_TYPE * EMBED)
    out = _expand(idx, table)
    return out.reshape(BATCH, HIST, EMBED)


# speedup vs baseline: 8.7907x; 1.0064x over previous
import functools

import jax
import jax.numpy as jnp
from jax import lax
from jax.experimental import pallas as pl
from jax.experimental.pallas import tpu as pltpu
from jax.experimental.pallas import tpu_sc as plsc

BATCH = 4096
HIST = 200
EMBED = 128
NUM_EDGE_TYPE = 64
N_ROWS = BATCH * HIST
NUM_WORKERS = 32
ROWS_PER_W = N_ROWS // NUM_WORKERS
CHUNK = 128
N_CHUNKS = ROWS_PER_W // CHUNK
NBUF = 4
COLB = EMBED // 16

_mesh = plsc.VectorSubcoreMesh(core_axis_name="c", subcore_axis_name="s")


@functools.partial(
    pl.kernel,
    mesh=_mesh,
    out_type=jax.ShapeDtypeStruct((N_ROWS * EMBED,), jnp.float32),
    compiler_params=pltpu.CompilerParams(needs_layout_passes=False),
    scratch_types=(
        [pltpu.VMEM((NUM_EDGE_TYPE * EMBED,), jnp.float32),
         pltpu.VMEM((NBUF * CHUNK,), jnp.int32),
         pltpu.VMEM((NBUF * CHUNK * EMBED,), jnp.float32)]
        + [pltpu.SemaphoreType.DMA] * (2 * NBUF)
    ),
)
def _expand(idx_hbm, table_hbm, out_hbm, table_v, idx_v, stage_v, *sems):
    isems, osems = sems[:NBUF], sems[NBUF:]
    wid = lax.axis_index("s") * 2 + lax.axis_index("c")
    crow_base = wid * N_CHUNKS
    out_base = wid * ROWS_PER_W * EMBED

    pltpu.sync_copy(table_hbm, table_v)

    def idesc(c, b):
        return pltpu.make_async_copy(
            idx_hbm.at[pl.ds((crow_base + c) * CHUNK, CHUNK)],
            idx_v.at[pl.ds(b * CHUNK, CHUNK)],
            isems[b])

    def odesc(c, b):
        return pltpu.make_async_copy(
            stage_v.at[pl.ds(b * CHUNK * EMBED, CHUNK * EMBED)],
            out_hbm.at[pl.ds(out_base + c * CHUNK * EMBED, CHUNK * EMBED)],
            osems[b])

    def expand(b):
        @plsc.parallel_loop(0, CHUNK // 16, unroll=2)
        def qbody(q):
            ivec = idx_v[pl.ds(b * CHUNK + q * 16, 16)]
            dst0 = (b * CHUNK + q * 16) * EMBED
            for l in range(16):
                src = ivec[l] * EMBED
                dst = dst0 + l * EMBED
                for j in range(COLB):
                    stage_v[pl.ds(dst + 16 * j, 16)] = (
                        table_v[pl.ds(src + 16 * j, 16)])

    for b in range(NBUF):
        idesc(b, b).start()

    n_groups = N_CHUNKS // NBUF

    def body(g, carry):
        cb = NBUF * g
        for b in range(NBUF):
            c = cb + b
            pl.when(g > 0)(lambda: odesc(c - NBUF, b).wait())
            idesc(c, b).wait()
            expand(b)
            pl.when(g < n_groups - 1)(lambda: idesc(c + NBUF, b).start())
            odesc(c, b).start()
        return carry

    lax.fori_loop(0, n_groups, body, 0)
    for b in range(NBUF):
        odesc(N_CHUNKS - NBUF + b, b).wait()


def kernel(data, edge_type_embedding):
    idx = data.reshape(N_ROWS)
    table = edge_type_embedding.reshape(NUM_EDGE_TYPE * EMBED)
    out = _expand(idx, table)
    return out.reshape(BATCH, HIST, EMBED)


# speedup vs baseline: 20.1020x; 2.2867x over previous
import functools

import jax
import jax.numpy as jnp
from jax import lax
from jax.experimental import pallas as pl
from jax.experimental.pallas import tpu as pltpu
from jax.experimental.pallas import tpu_sc as plsc

BATCH = 4096
HIST = 200
EMBED = 128
NUM_EDGE_TYPE = 64
N_ROWS = BATCH * HIST
NUM_WORKERS = 32
ROWS_PER_W = N_ROWS // NUM_WORKERS
CHUNK = 128
N_CHUNKS = ROWS_PER_W // CHUNK
NBUF = 4

_mesh = plsc.VectorSubcoreMesh(core_axis_name="c", subcore_axis_name="s")


@functools.partial(
    pl.kernel,
    mesh=_mesh,
    out_type=jax.ShapeDtypeStruct((N_ROWS, EMBED), jnp.float32),
    compiler_params=pltpu.CompilerParams(needs_layout_passes=False),
    scratch_types=(
        [pltpu.VMEM_SHARED((NUM_EDGE_TYPE, EMBED), jnp.float32),
         pltpu.VMEM((NBUF, CHUNK), jnp.int32),
         pltpu.VMEM((NBUF * CHUNK, EMBED), jnp.float32)]
        + [pltpu.SemaphoreType.DMA] * (2 * NBUF)
    ),
)
def _gather(idx_hbm, table_hbm, out_hbm, table_sh, idx_v, rows_v, *sems):
    gsems, osems = sems[:NBUF], sems[NBUF:]
    sid = lax.axis_index("s")
    wid = sid * 2 + lax.axis_index("c")
    irow_base = wid * N_CHUNKS
    out_base = wid * ROWS_PER_W

    pl.when(sid == 0)(lambda: pltpu.sync_copy(table_hbm, table_sh))
    plsc.subcore_barrier()

    def load_idx(c, b):
        pltpu.sync_copy(idx_hbm.at[pl.ds(irow_base + c, 1)],
                        idx_v.at[pl.ds(b, 1)])

    def gdesc(b):
        return pltpu.make_async_copy(
            table_sh.at[idx_v.at[b]],
            rows_v.at[pl.ds(b * CHUNK, CHUNK)],
            gsems[b])

    def odesc(c, b):
        return pltpu.make_async_copy(
            rows_v.at[pl.ds(b * CHUNK, CHUNK)],
            out_hbm.at[pl.ds(out_base + c * CHUNK, CHUNK)],
            osems[b])

    load_idx(0, 0); gdesc(0).start()
    load_idx(1, 1); gdesc(1).start()
    load_idx(2, 2); gdesc(2).start()
    gdesc(0).wait(); odesc(0, 0).start()
    load_idx(3, 3); gdesc(3).start()
    gdesc(1).wait(); odesc(1, 1).start()

    def body(g, carry):
        cb = 4 * g + 4
        for b in range(NBUF):
            c = cb + b
            odesc(c - 4, b).wait()
            load_idx(c, b)
            gdesc(b).start()
            b2 = (b + 2) % NBUF
            gdesc(b2).wait()
            odesc(c - 2, b2).start()
        return carry

    lax.fori_loop(0, (N_CHUNKS - 4) // NBUF, body, 0)

    gdesc(2).wait(); odesc(N_CHUNKS - 2, 2).start()
    gdesc(3).wait(); odesc(N_CHUNKS - 1, 3).start()
    for b in range(NBUF):
        odesc(N_CHUNKS - 4 + b, b).wait()


def kernel(data, edge_type_embedding):
    idx = data.reshape(N_ROWS // CHUNK, CHUNK)
    out = _gather(idx, edge_type_embedding)
    return out.reshape(BATCH, HIST, EMBED)


# speedup vs baseline: 20.4982x; 1.0197x over previous
import functools

import jax
import jax.numpy as jnp
from jax import lax
from jax.experimental import pallas as pl
from jax.experimental.pallas import tpu as pltpu
from jax.experimental.pallas import tpu_sc as plsc

BATCH = 4096
HIST = 200
EMBED = 128
NUM_EDGE_TYPE = 64
N_ROWS = BATCH * HIST
NUM_WORKERS = 32
ROWS_PER_W = N_ROWS // NUM_WORKERS
CHUNK = 128
N_CHUNKS = ROWS_PER_W // CHUNK
NBUF = 4

_mesh = plsc.VectorSubcoreMesh(core_axis_name="c", subcore_axis_name="s")


@functools.partial(
    pl.kernel,
    mesh=_mesh,
    out_type=jax.ShapeDtypeStruct((N_ROWS, EMBED), jnp.float32),
    compiler_params=pltpu.CompilerParams(needs_layout_passes=False),
    scratch_types=(
        [pltpu.VMEM_SHARED((NUM_EDGE_TYPE, EMBED), jnp.float32),
         pltpu.VMEM((NBUF, CHUNK), jnp.int32),
         pltpu.VMEM((NBUF * CHUNK, EMBED), jnp.float32)]
        + [pltpu.SemaphoreType.DMA] * (3 * NBUF)
    ),
)
def _gather(idx_hbm, table_hbm, out_hbm, table_sh, idx_v, rows_v, *sems):
    gsems, osems, isems = sems[:NBUF], sems[NBUF:2 * NBUF], sems[2 * NBUF:]
    sid = lax.axis_index("s")
    wid = sid * 2 + lax.axis_index("c")
    irow_base = wid * N_CHUNKS
    out_base = wid * ROWS_PER_W

    pl.when(sid == 0)(lambda: pltpu.sync_copy(table_hbm, table_sh))
    plsc.subcore_barrier()

    def idesc(c, b):
        return pltpu.make_async_copy(
            idx_hbm.at[pl.ds(irow_base + c, 1)],
            idx_v.at[pl.ds(b, 1)],
            isems[b])

    def gdesc(b):
        return pltpu.make_async_copy(
            table_sh.at[idx_v.at[b]],
            rows_v.at[pl.ds(b * CHUNK, CHUNK)],
            gsems[b])

    def odesc(c, b):
        return pltpu.make_async_copy(
            rows_v.at[pl.ds(b * CHUNK, CHUNK)],
            out_hbm.at[pl.ds(out_base + c * CHUNK, CHUNK)],
            osems[b])

    for b in range(NBUF):
        idesc(b, b).start()

    n_groups = N_CHUNKS // NBUF

    def body(g, carry):
        cb = NBUF * g
        for b in range(NBUF):
            c = cb + b
            pl.when(g > 0)(lambda: odesc(c - NBUF, b).wait())
            idesc(c, b).wait()
            gdesc(b).start()
            b2 = (b + 2) % NBUF
            c2 = c - 2

            def ship():
                gdesc(b2).wait()
                odesc(c2, b2).start()

            def prefetch():
                idesc(c2 + NBUF, b2).start()

            pl.when(c >= 2)(ship)
            pl.when((c >= 2) & (c2 + NBUF < N_CHUNKS))(prefetch)
        return carry

    lax.fori_loop(0, n_groups, body, 0)

    gdesc(2).wait(); odesc(N_CHUNKS - 2, 2).start()
    gdesc(3).wait(); odesc(N_CHUNKS - 1, 3).start()
    for b in range(NBUF):
        odesc(N_CHUNKS - NBUF + b, b).wait()


def kernel(data, edge_type_embedding):
    idx = data.reshape(N_ROWS // CHUNK, CHUNK)
    out = _gather(idx, edge_type_embedding)
    return out.reshape(BATCH, HIST, EMBED)


# speedup vs baseline: 20.4991x; 1.0000x over previous
import functools

import jax
import jax.numpy as jnp
from jax import lax
from jax.experimental import pallas as pl
from jax.experimental.pallas import tpu as pltpu
from jax.experimental.pallas import tpu_sc as plsc

BATCH = 4096
HIST = 200
EMBED = 128
NUM_EDGE_TYPE = 64
N_ROWS = BATCH * HIST
NUM_WORKERS = 32
ROWS_PER_W = N_ROWS // NUM_WORKERS
CHUNK = 128
N_CHUNKS = ROWS_PER_W // CHUNK
NBUF = 5

_mesh = plsc.VectorSubcoreMesh(core_axis_name="c", subcore_axis_name="s")


@functools.partial(
    pl.kernel,
    mesh=_mesh,
    out_type=jax.ShapeDtypeStruct((N_ROWS, EMBED), jnp.float32),
    compiler_params=pltpu.CompilerParams(needs_layout_passes=False),
    scratch_types=(
        [pltpu.VMEM_SHARED((NUM_EDGE_TYPE, EMBED), jnp.float32),
         pltpu.VMEM((NBUF, CHUNK), jnp.int32),
         pltpu.VMEM((NBUF * CHUNK, EMBED), jnp.float32)]
        + [pltpu.SemaphoreType.DMA] * (3 * NBUF)
    ),
)
def _gather(idx_hbm, table_hbm, out_hbm, table_sh, idx_v, rows_v, *sems):
    gsems, osems, isems = sems[:NBUF], sems[NBUF:2 * NBUF], sems[2 * NBUF:]
    sid = lax.axis_index("s")
    wid = sid * 2 + lax.axis_index("c")
    irow_base = wid * N_CHUNKS
    out_base = wid * ROWS_PER_W

    pl.when(sid == 0)(lambda: pltpu.sync_copy(table_hbm, table_sh))
    plsc.subcore_barrier()

    def idesc(c, b):
        return pltpu.make_async_copy(
            idx_hbm.at[pl.ds(irow_base + c, 1)],
            idx_v.at[pl.ds(b, 1)],
            isems[b])

    def gdesc(b):
        return pltpu.make_async_copy(
            table_sh.at[idx_v.at[b]],
            rows_v.at[pl.ds(b * CHUNK, CHUNK)],
            gsems[b])

    def odesc(c, b):
        return pltpu.make_async_copy(
            rows_v.at[pl.ds(b * CHUNK, CHUNK)],
            out_hbm.at[pl.ds(out_base + c * CHUNK, CHUNK)],
            osems[b])

    for b in range(NBUF):
        idesc(b, b).start()

    n_groups = N_CHUNKS // NBUF

    def body(g, carry):
        cb = NBUF * g
        for b in range(NBUF):
            c = cb + b
            pl.when(g > 0)(lambda: odesc(c - NBUF, b).wait())
            idesc(c, b).wait()
            gdesc(b).start()
            b2 = (b - 2) % NBUF
            c2 = c - 2

            def ship():
                gdesc(b2).wait()
                odesc(c2, b2).start()

            def prefetch():
                idesc(c2 + NBUF, b2).start()

            pl.when(c >= 2)(ship)
            pl.when((c >= 2) & (c2 + NBUF < N_CHUNKS))(prefetch)
        return carry

    lax.fori_loop(0, n_groups, body, 0)

    for c in (N_CHUNKS - 2, N_CHUNKS - 1):
        gdesc(c % NBUF).wait()
        odesc(c, c % NBUF).start()
    for c in range(N_CHUNKS - NBUF, N_CHUNKS):
        odesc(c, c % NBUF).wait()


def kernel(data, edge_type_embedding):
    idx = data.reshape(N_ROWS // CHUNK, CHUNK)
    out = _gather(idx, edge_type_embedding)
    return out.reshape(BATCH, HIST, EMBED)
